# baseline (speedup 1.0000x reference)
import functools

import jax
import jax.numpy as jnp
from jax import lax
from jax.experimental import pallas as pl
from jax.experimental.pallas import tpu as pltpu
from jax.experimental.pallas import tpu_sc as plsc

VOCAB = 1000000
DIM = 64
EPS = 1e-06

NC = 2
NS = 16
NW = NC * NS

TOTAL_ROWS = 4096 * 200
ROWS_PER_TILE = TOTAL_ROWS // NW
CHUNK = 256
G = CHUNK // 128
N_CHUNKS = ROWS_PER_TILE // CHUNK
IDS_ROWS = TOTAL_ROWS // 128
NQ = DIM // 16


def _permute(x, idx):
    return lax.gather(
        x,
        idx[:, None],
        dimension_numbers=lax.GatherDimensionNumbers(
            offset_dims=(), collapsed_slice_dims=(0,), start_index_map=(0,)
        ),
        slice_sizes=(1,),
        mode=lax.GatherScatterMode.PROMISE_IN_BOUNDS,
    )


def _rsqrt(x):
    i = lax.bitcast_convert_type(x, jnp.int32)
    i = jnp.int32(0x5F3759DF) - lax.shift_right_logical(i, 1)
    y = lax.bitcast_convert_type(i, jnp.float32)
    for _ in range(2):
        y = y * (1.5 - 0.5 * x * y * y)
    return y


@functools.partial(
    pl.kernel,
    mesh=plsc.VectorSubcoreMesh(core_axis_name="c", subcore_axis_name="s"),
    out_type=jax.ShapeDtypeStruct((TOTAL_ROWS, 128), jnp.float32),
    compiler_params=pltpu.CompilerParams(use_tc_tiling_on_sc=False),
    scratch_types=[
        pltpu.VMEM((G, 128), jnp.int32),
        pltpu.VMEM((G, 128), jnp.int32),
        pltpu.VMEM((G, 128), jnp.int32),
        pltpu.VMEM((CHUNK, 128), jnp.float32),
        pltpu.VMEM((CHUNK, 128), jnp.float32),
        pltpu.VMEM((CHUNK, 128), jnp.float32),
        pltpu.VMEM((DIM,), jnp.float32),
        pltpu.SemaphoreType.DMA,
        pltpu.SemaphoreType.DMA,
        pltpu.SemaphoreType.DMA,
        pltpu.SemaphoreType.DMA,
        pltpu.SemaphoreType.DMA,
        pltpu.SemaphoreType.DMA,
    ],
)
def _sc_embed_rmsnorm(
    ids_hbm, tab_hbm, nw_hbm, out_hbm,
    idx0, idx1, idx2, buf0, buf1, buf2, nw_v,
    gsem0, gsem1, gsem2, osem0, osem1, osem2,
):
    wid = lax.axis_index("s") * NC + lax.axis_index("c")
    row_base = wid * ROWS_PER_TILE
    irow_base = wid * (ROWS_PER_TILE // 128)

    pltpu.sync_copy(nw_hbm, nw_v)
    w_vecs = [nw_v[pl.ds(q * 16, 16)] * float(DIM**0.5) for q in range(NQ)]
    lanes = lax.iota(jnp.int32, 16)

    idx_bufs = (idx0, idx1, idx2)
    bufs = (buf0, buf1, buf2)
    gsems = (gsem0, gsem1, gsem2)
    osems = (osem0, osem1, osem2)

    def copy_idx(c, k):
        pltpu.sync_copy(ids_hbm.at[pl.ds(irow_base + c * G, G)], idx_bufs[k])

    def fire_gathers(k):
        for g in range(G):
            pltpu.async_copy(
                tab_hbm.at[idx_bufs[k].at[g]],
                bufs[k].at[pl.ds(g * 128, 128)],
                gsems[k],
            )

    def drain_gathers(k):
        for g in range(G):
            pltpu.make_async_copy(
                tab_hbm.at[idx_bufs[k].at[g]],
                bufs[k].at[pl.ds(g * 128, 128)],
                gsems[k],
            ).wait()

    def fire_scatter(c, k):
        pltpu.async_copy(
            bufs[k].at[:, pl.ds(0, DIM)],
            out_hbm.at[pl.ds(row_base + c * CHUNK, CHUNK), pl.ds(0, DIM)],
            osems[k],
        )

    def wait_scatter(c, k):
        pltpu.make_async_copy(
            bufs[k].at[:, pl.ds(0, DIM)],
            out_hbm.at[pl.ds(row_base + c * CHUNK, CHUNK), pl.ds(0, DIM)],
            osems[k],
        ).wait()

    def compute(k):
        rows_v = bufs[k]

        def row_body(j, _):
            for u in range(4):
                r = j * 4 + u
                q = [rows_v[r, pl.ds(16 * i, 16)] for i in range(NQ)]
                t = q[0] * q[0]
                for qi in q[1:]:
                    t = t + qi * qi
                for kk in (1, 2, 4, 8):
                    t = t + _permute(t, lanes ^ kk)
                scale = _rsqrt(t + DIM * EPS)
                for i in range(NQ):
                    rows_v[r, pl.ds(16 * i, 16)] = q[i] * scale * w_vecs[i]
            return 0

        lax.fori_loop(0, CHUNK // 4, row_body, 0)

    def half(c, k3, *, wait_prev_scatter, fire_next, copy_next_idx):
        if wait_prev_scatter:
            wait_scatter(c - 2, (k3 + 1) % 3)
        if fire_next:
            fire_gathers((k3 + 1) % 3)
        drain_gathers(k3)
        if copy_next_idx:
            copy_idx(c + 2, (k3 + 2) % 3)
        compute(k3)
        fire_scatter(c, k3)

    copy_idx(0, 0)
    copy_idx(1, 1)
    fire_gathers(0)
    half(0, 0, wait_prev_scatter=False, fire_next=True, copy_next_idx=True)
    half(1, 1, wait_prev_scatter=False, fire_next=True, copy_next_idx=True)
    half(2, 2, wait_prev_scatter=True, fire_next=True, copy_next_idx=True)

    def triple_body(j, _):
        c0 = j * 3
        half(c0, 0, wait_prev_scatter=True, fire_next=True, copy_next_idx=True)
        half(c0 + 1, 1, wait_prev_scatter=True, fire_next=True, copy_next_idx=True)
        half(c0 + 2, 2, wait_prev_scatter=True, fire_next=True, copy_next_idx=True)
        return 0

    n_triples = (N_CHUNKS - 3 - 2) // 3
    lax.fori_loop(1, 1 + n_triples, triple_body, 0)

    for c in range(3 + 3 * n_triples, N_CHUNKS):
        half(c, c % 3, wait_prev_scatter=True,
             fire_next=(c + 1 < N_CHUNKS),
             copy_next_idx=(c + 2 < N_CHUNKS))
    wait_scatter(N_CHUNKS - 2, (N_CHUNKS - 2) % 3)
    wait_scatter(N_CHUNKS - 1, (N_CHUNKS - 1) % 3)


def kernel(input_ids, table, norm_weight):
    b, l = input_ids.shape
    ids = input_ids.reshape(IDS_ROWS, 128).astype(jnp.int32)
    tab2 = jnp.pad(table, ((0, 0), (0, 128 - DIM)))
    out = _sc_embed_rmsnorm(ids, tab2, norm_weight)
    return out.reshape(b, l, 128)[:, :, :DIM]


# speedup vs baseline: 1.1266x; 1.1266x over previous
import functools

import jax
import jax.numpy as jnp
from jax import lax
from jax.experimental import pallas as pl
from jax.experimental.pallas import tpu as pltpu
from jax.experimental.pallas import tpu_sc as plsc

VOCAB = 1000000
DIM = 64
EPS = 1e-06

NC = 2
NS = 16
NW = NC * NS

TOTAL_ROWS = 4096 * 200
ROWS_PER_TILE = TOTAL_ROWS // NW
CHUNK = 512
G = CHUNK // 128
N_CHUNKS = ROWS_PER_TILE // CHUNK
IDS_ROWS = TOTAL_ROWS // 128
NQ = DIM // 16


def _permute(x, idx):
    return lax.gather(
        x,
        idx[:, None],
        dimension_numbers=lax.GatherDimensionNumbers(
            offset_dims=(), collapsed_slice_dims=(0,), start_index_map=(0,)
        ),
        slice_sizes=(1,),
        mode=lax.GatherScatterMode.PROMISE_IN_BOUNDS,
    )


def _rsqrt(x):
    i = lax.bitcast_convert_type(x, jnp.int32)
    i = jnp.int32(0x5F3759DF) - lax.shift_right_logical(i, 1)
    y = lax.bitcast_convert_type(i, jnp.float32)
    for _ in range(2):
        y = y * (1.5 - 0.5 * x * y * y)
    return y


@functools.partial(
    pl.kernel,
    mesh=plsc.VectorSubcoreMesh(core_axis_name="c", subcore_axis_name="s"),
    out_type=jax.ShapeDtypeStruct((TOTAL_ROWS, 128), jnp.float32),
    compiler_params=pltpu.CompilerParams(use_tc_tiling_on_sc=False),
    scratch_types=[
        pltpu.VMEM((G, 128), jnp.int32),
        pltpu.VMEM((G, 128), jnp.int32),
        pltpu.VMEM((G, 128), jnp.int32),
        pltpu.VMEM((CHUNK, DIM), jnp.float32),
        pltpu.VMEM((CHUNK, DIM), jnp.float32),
        pltpu.VMEM((CHUNK, DIM), jnp.float32),
        pltpu.VMEM((DIM,), jnp.float32),
        pltpu.SemaphoreType.DMA,
        pltpu.SemaphoreType.DMA,
        pltpu.SemaphoreType.DMA,
        pltpu.SemaphoreType.DMA,
        pltpu.SemaphoreType.DMA,
        pltpu.SemaphoreType.DMA,
        pltpu.SemaphoreType.DMA,
        pltpu.SemaphoreType.DMA,
        pltpu.SemaphoreType.DMA,
    ],
)
def _sc_embed_rmsnorm(
    ids_hbm, tab_hbm, nw_hbm, out_hbm,
    idx0, idx1, idx2, buf0, buf1, buf2, nw_v,
    gsem0, gsem1, gsem2, osem0, osem1, osem2, isem0, isem1, isem2,
):
    wid = lax.axis_index("s") * NC + lax.axis_index("c")
    row_base = wid * ROWS_PER_TILE
    irow_base = wid * (ROWS_PER_TILE // 128)

    pltpu.sync_copy(nw_hbm, nw_v)
    w_vecs = [nw_v[pl.ds(q * 16, 16)] * float(DIM**0.5) for q in range(NQ)]
    lanes = lax.iota(jnp.int32, 16)

    idx_bufs = (idx0, idx1, idx2)
    bufs = (buf0, buf1, buf2)
    gsems = (gsem0, gsem1, gsem2)
    osems = (osem0, osem1, osem2)
    isems = (isem0, isem1, isem2)

    def copy_idx(c, k):
        pltpu.sync_copy(ids_hbm.at[pl.ds(irow_base + c * G, G)], idx_bufs[k])

    def fire_idx(c, k):
        pltpu.async_copy(
            ids_hbm.at[pl.ds(irow_base + c * G, G)], idx_bufs[k], isems[k]
        )

    def wait_idx(k):
        pltpu.make_async_copy(
            ids_hbm.at[pl.ds(0, G)], idx_bufs[k], isems[k]
        ).wait()

    def fire_gathers(k):
        for g in range(G):
            pltpu.async_copy(
                tab_hbm.at[idx_bufs[k].at[g]],
                bufs[k].at[pl.ds(g * 128, 128)],
                gsems[k],
            )

    def drain_gathers(k):
        for g in range(G):
            pltpu.make_async_copy(
                tab_hbm.at[idx_bufs[k].at[g]],
                bufs[k].at[pl.ds(g * 128, 128)],
                gsems[k],
            ).wait()

    def fire_scatter(c, k):
        pltpu.async_copy(
            bufs[k],
            out_hbm.at[pl.ds(row_base + c * CHUNK, CHUNK), pl.ds(0, DIM)],
            osems[k],
        )

    def wait_scatter(c, k):
        pltpu.make_async_copy(
            bufs[k],
            out_hbm.at[pl.ds(row_base + c * CHUNK, CHUNK), pl.ds(0, DIM)],
            osems[k],
        ).wait()

    def compute(k):
        rows_v = bufs[k]

        def row_body(j, _):
            for u in range(4):
                r = j * 4 + u
                q = [rows_v[r, pl.ds(16 * i, 16)] for i in range(NQ)]
                t = q[0] * q[0]
                for qi in q[1:]:
                    t = t + qi * qi
                for kk in (1, 2, 4, 8):
                    t = t + _permute(t, lanes ^ kk)
                scale = _rsqrt(t + DIM * EPS)
                for i in range(NQ):
                    rows_v[r, pl.ds(16 * i, 16)] = q[i] * scale * w_vecs[i]
            return 0

        lax.fori_loop(0, CHUNK // 4, row_body, 0)

    def half(c, k3, *, wait_prev_scatter, fire_next, copy_next_idx,
             idx_is_async=True):
        if wait_prev_scatter:
            wait_scatter(c - 2, (k3 + 1) % 3)
        if fire_next:
            if idx_is_async:
                wait_idx((k3 + 1) % 3)
            fire_gathers((k3 + 1) % 3)
        drain_gathers(k3)
        if copy_next_idx:
            fire_idx(c + 2, (k3 + 2) % 3)
        compute(k3)
        fire_scatter(c, k3)

    copy_idx(0, 0)
    copy_idx(1, 1)
    fire_gathers(0)
    half(0, 0, wait_prev_scatter=False, fire_next=True, copy_next_idx=True,
         idx_is_async=False)
    half(1, 1, wait_prev_scatter=False, fire_next=True, copy_next_idx=True)
    half(2, 2, wait_prev_scatter=True, fire_next=True, copy_next_idx=True)

    def triple_body(j, _):
        c0 = j * 3
        half(c0, 0, wait_prev_scatter=True, fire_next=True, copy_next_idx=True)
        half(c0 + 1, 1, wait_prev_scatter=True, fire_next=True, copy_next_idx=True)
        half(c0 + 2, 2, wait_prev_scatter=True, fire_next=True, copy_next_idx=True)
        return 0

    n_triples = (N_CHUNKS - 3 - 2) // 3
    lax.fori_loop(1, 1 + n_triples, triple_body, 0)

    for c in range(3 + 3 * n_triples, N_CHUNKS):
        half(c, c % 3, wait_prev_scatter=True,
             fire_next=(c + 1 < N_CHUNKS),
             copy_next_idx=(c + 2 < N_CHUNKS))
    wait_scatter(N_CHUNKS - 2, (N_CHUNKS - 2) % 3)
    wait_scatter(N_CHUNKS - 1, (N_CHUNKS - 1) % 3)


def kernel(input_ids, table, norm_weight):
    b, l = input_ids.shape
    ids = input_ids.reshape(IDS_ROWS, 128).astype(jnp.int32)
    out = _sc_embed_rmsnorm(ids, table, norm_weight)
    return out.reshape(b, l, 128)[:, :, :DIM]
